# Initial kernel scaffold; baseline (speedup 1.0000x reference)
import jax
import jax.numpy as jnp
from jax.experimental import pallas as pl


def kernel(x, pos, W_pos1, b_pos1, W_pos2, b_pos2, W_attn1, b_attn1, W_attn2, b_attn2, Wq, Wk, Wv, Wo, bo):
    raise NotImplementedError("write your pallas kernel here")


# speedup vs baseline: 12.4625x; 12.4625x over previous
import functools

import jax
import jax.numpy as jnp
from jax import lax
from jax.experimental import pallas as pl
from jax.experimental.pallas import tpu as pltpu
from jax.experimental.pallas import tpu_sc as plsc

K = 16
ROW = 128
BLK_A = 256
BLK_C = 256
NC, NS = 2, 16
CH = 128


def _topk_body(posq_ref, post_ref, sq_ref, idx_ref):
    b = pl.program_id(0)
    posq = posq_ref[0]
    post = post_ref[0]
    sq = sq_ref[0]
    n = post.shape[1]
    s = lax.dot_general(posq, post, (((1,), (0,)), ((), ())),
                        preferred_element_type=jnp.float32)
    scores = s + s - sq
    iota = lax.broadcasted_iota(jnp.int32, scores.shape, 1)
    neg = jnp.float32(-3.0e38)
    cols = []
    for _ in range(K):
        m = jnp.max(scores, axis=1, keepdims=True)
        eq = scores == m
        idxk = jnp.min(jnp.where(eq, iota, n), axis=1, keepdims=True)
        cols.append(idxk)
        scores = jnp.where(iota == idxk, neg, scores)
    idx = jnp.concatenate(cols, axis=1)
    idx_ref[0] = idx + b * n


def _topk(pos):
    b, n, _ = pos.shape
    post = jnp.transpose(pos, (0, 2, 1))
    sq = jnp.sum(pos ** 2, axis=2)[:, None, :]
    return pl.pallas_call(
        _topk_body,
        grid=(b, n // BLK_A),
        in_specs=[
            pl.BlockSpec((1, BLK_A, 3), lambda bi, i: (bi, i, 0)),
            pl.BlockSpec((1, 3, n), lambda bi, i: (bi, 0, 0)),
            pl.BlockSpec((1, 1, n), lambda bi, i: (bi, 0, 0)),
        ],
        out_specs=pl.BlockSpec((1, BLK_A, K), lambda bi, i: (bi, i, 0)),
        out_shape=jax.ShapeDtypeStruct((b, n, K), jnp.int32),
    )(pos, post, sq)


def _gather_body(tab_ref, idx_ref, out_ref, idx_v, rows_v, sem):
    wid = lax.axis_index("s") * NC + lax.axis_index("c")
    per_w = idx_ref.shape[0] // (NC * NS)
    base = wid * per_w

    def step(i, carry):
        off = pl.multiple_of(base + i * CH, 8)
        pltpu.sync_copy(idx_ref.at[pl.ds(off, CH)], idx_v)
        pltpu.async_copy(tab_ref.at[idx_v], rows_v, sem).wait()
        pltpu.sync_copy(rows_v, out_ref.at[pl.ds(off, CH)])
        return carry

    lax.fori_loop(0, per_w // CH, step, 0)


def _gather(tab, idx_flat):
    bnk = idx_flat.shape[0]
    run = functools.partial(
        pl.kernel,
        mesh=plsc.VectorSubcoreMesh(core_axis_name="c", subcore_axis_name="s"),
        out_type=jax.ShapeDtypeStruct((bnk, ROW), jnp.float32),
        scratch_types=[
            pltpu.VMEM((CH,), jnp.int32),
            pltpu.VMEM((CH, ROW), jnp.float32),
            pltpu.SemaphoreType.DMA,
        ],
    )(_gather_body)
    return run(tab, idx_flat)


def _attn_body(x_ref, pos_ref, g_ref, wp1_ref, bp1_ref, wp2_ref, bp2_ref,
               wa1_ref, ba1_ref, wa2_ref, ba2_ref, wq_ref, wk_ref, wv_ref,
               wo_ref, bo_ref, out_ref):
    xq = x_ref[0]
    posq = pos_ref[0]
    g = g_ref[0]
    d = xq.shape[1]
    wp1 = wp1_ref[...]
    bp1 = bp1_ref[...]
    bp2 = bp2_ref[...]
    ba1 = ba1_ref[...]
    ba2 = ba2_ref[...]
    q = jnp.dot(xq, wq_ref[...], preferred_element_type=jnp.float32)
    a_list, v_list = [], []
    for k in range(K):
        f = g[:, k * ROW:k * ROW + d]
        p3 = g[:, k * ROW + d:k * ROW + d + 3]
        rel = p3 - posq
        pe = (rel[:, 0:1] * wp1[0:1, :] + rel[:, 1:2] * wp1[1:2, :]
              + rel[:, 2:3] * wp1[2:3, :] + bp1)
        pe = jnp.maximum(pe, 0.0)
        pe = jnp.dot(pe, wp2_ref[...], preferred_element_type=jnp.float32) + bp2
        kk = jnp.dot(f, wk_ref[...], preferred_element_type=jnp.float32)
        vv = jnp.dot(f, wv_ref[...], preferred_element_type=jnp.float32)
        e = q - kk + pe
        a = jnp.maximum(
            jnp.dot(e, wa1_ref[...], preferred_element_type=jnp.float32) + ba1,
            0.0)
        a = jnp.dot(a, wa2_ref[...], preferred_element_type=jnp.float32) + ba2
        a_list.append(a)
        v_list.append(vv)
    mx = a_list[0]
    for a in a_list[1:]:
        mx = jnp.maximum(mx, a)
    z = None
    o = None
    for a, vv in zip(a_list, v_list):
        s = jnp.exp(a - mx)
        z = s if z is None else z + s
        o = s * vv if o is None else o + s * vv
    o = o / z
    out_ref[0] = jnp.dot(o, wo_ref[...], preferred_element_type=jnp.float32) \
        + bo_ref[...]


def _attn(x, pos, g3, wp1, bp1, wp2, bp2, wa1, ba1, wa2, ba2, wq, wk, wv,
          wo, bo):
    b, n, d = x.shape
    full = lambda a: pl.BlockSpec(a.shape, lambda bi, i: (0,) * a.ndim)
    return pl.pallas_call(
        _attn_body,
        grid=(b, n // BLK_C),
        in_specs=[
            pl.BlockSpec((1, BLK_C, d), lambda bi, i: (bi, i, 0)),
            pl.BlockSpec((1, BLK_C, 3), lambda bi, i: (bi, i, 0)),
            pl.BlockSpec((1, BLK_C, K * ROW), lambda bi, i: (bi, i, 0)),
            full(wp1), full(bp1), full(wp2), full(bp2),
            full(wa1), full(ba1), full(wa2), full(ba2),
            full(wq), full(wk), full(wv), full(wo), full(bo),
        ],
        out_specs=pl.BlockSpec((1, BLK_C, d), lambda bi, i: (bi, i, 0)),
        out_shape=jax.ShapeDtypeStruct((b, n, d), jnp.float32),
    )(x, pos, g3, wp1, bp1, wp2, bp2, wa1, ba1, wa2, ba2, wq, wk, wv, wo, bo)


def kernel(x, pos, W_pos1, b_pos1, W_pos2, b_pos2, W_attn1, b_attn1,
           W_attn2, b_attn2, Wq, Wk, Wv, Wo, bo):
    b, n, d = x.shape
    idx = _topk(pos)
    pad = jnp.zeros((b, n, ROW - d - 3), jnp.float32)
    tab = jnp.concatenate([x, pos, pad], axis=-1).reshape(b * n, ROW)
    g = _gather(tab, idx.reshape(-1))
    g3 = g.reshape(b, n, K * ROW)
    return _attn(x, pos, g3, W_pos1, b_pos1.reshape(1, d), W_pos2,
                 b_pos2.reshape(1, d), W_attn1, b_attn1.reshape(1, d),
                 W_attn2, b_attn2.reshape(1, d), Wq, Wk, Wv, Wo,
                 bo.reshape(1, d))
